# baseline (speedup 1.0000x reference)
import functools

import jax
import jax.numpy as jnp
from jax import lax
from jax.experimental import pallas as pl
from jax.experimental.pallas import tpu as pltpu
from jax.experimental.pallas import tpu_sc as plsc

NH = 8
NP = 4
K = NH * NP
NC = 2
NS = 16


def _corners_body(q_ref, rp_ref, wx_ref, wy_ref, bx_ref, by_ref, wa_ref,
                  ba_ref, oi_ref, ov_ref, *, W, H, L):
    dn = (((1,), (1,)), ((), ()))
    q = q_ref[0]
    rp = rp_ref[0]

    offx = jax.lax.dot_general(q, wx_ref[...], dn,
                               preferred_element_type=jnp.float32)
    offx = offx + bx_ref[...]
    offy = jax.lax.dot_general(q, wy_ref[...], dn,
                               preferred_element_type=jnp.float32)
    offy = offy + by_ref[...]
    logits = jax.lax.dot_general(q, wa_ref[...], dn,
                                 preferred_element_type=jnp.float32)
    logits = logits + ba_ref[...]

    m = jnp.max(logits, axis=1, keepdims=True)
    ex = jnp.exp(logits - m)
    gi = jax.lax.broadcasted_iota(jnp.int32, (K, K), 0) // NP
    gj = jax.lax.broadcasted_iota(jnp.int32, (K, K), 1) // NP
    G = (gi == gj).astype(jnp.float32)
    denom = jnp.dot(ex, G, preferred_element_type=jnp.float32)
    aw = ex / denom * (1.0 / NH)

    locx = rp[:, 0:1] + offx
    locy = rp[:, 1:2] + offy
    ix = locx * float(W) - 0.5
    iy = locy * float(H) - 0.5

    x0f = jnp.floor(ix)
    y0f = jnp.floor(iy)
    fx = ix - x0f
    fy = iy - y0f
    x1f = x0f + 1.0
    y1f = y0f + 1.0

    vx0 = ((x0f >= 0.0) & (x0f <= W - 1.0)).astype(jnp.float32)
    vx1 = ((x1f >= 0.0) & (x1f <= W - 1.0)).astype(jnp.float32)
    vy0 = ((y0f >= 0.0) & (y0f <= H - 1.0)).astype(jnp.float32)
    vy1 = ((y1f >= 0.0) & (y1f <= H - 1.0)).astype(jnp.float32)

    x0 = jnp.clip(x0f, 0.0, W - 1.0).astype(jnp.int32)
    x1 = jnp.clip(x1f, 0.0, W - 1.0).astype(jnp.int32)
    y0 = jnp.clip(y0f, 0.0, H - 1.0).astype(jnp.int32)
    y1 = jnp.clip(y1f, 0.0, H - 1.0).astype(jnp.int32)

    base = jax.lax.broadcasted_iota(jnp.int32, (L, 1), 0) * (H * W)
    g00 = base + y0 * W + x0
    g01 = base + y0 * W + x1
    g10 = base + y1 * W + x0
    g11 = base + y1 * W + x1

    w00 = aw * (1.0 - fy) * (1.0 - fx) * (vy0 * vx0)
    w01 = aw * (1.0 - fy) * fx * (vy0 * vx1)
    w10 = aw * fy * (1.0 - fx) * (vy1 * vx0)
    w11 = aw * fy * fx * (vy1 * vx1)

    oi_ref[0] = jnp.concatenate([g00, g01, g10, g11], axis=1)
    ov_ref[0] = jnp.concatenate([w00, w01, w10, w11], axis=1)


def _corner_lists(q2, reference_points, W_off, b_off, W_attn, b_attn,
                  h, w, L, e):
    Wx = W_off[0::2]
    Wy = W_off[1::2]
    bx = b_off[0::2].reshape(1, K)
    by = b_off[1::2].reshape(1, K)
    ba = b_attn.reshape(1, K)
    n = q2.shape[0]
    idx, val = pl.pallas_call(
        functools.partial(_corners_body, W=w, H=h, L=L),
        grid=(n,),
        in_specs=[
            pl.BlockSpec((1, L, e), lambda i: (i, 0, 0)),
            pl.BlockSpec((1, L, 2), lambda i: (i, 0, 0)),
            pl.BlockSpec((K, e), lambda i: (0, 0)),
            pl.BlockSpec((K, e), lambda i: (0, 0)),
            pl.BlockSpec((1, K), lambda i: (0, 0)),
            pl.BlockSpec((1, K), lambda i: (0, 0)),
            pl.BlockSpec((K, e), lambda i: (0, 0)),
            pl.BlockSpec((1, K), lambda i: (0, 0)),
        ],
        out_specs=[
            pl.BlockSpec((1, L, 4 * K), lambda i: (i, 0, 0)),
            pl.BlockSpec((1, L, 4 * K), lambda i: (i, 0, 0)),
        ],
        out_shape=[
            jax.ShapeDtypeStruct((n, L, 4 * K), jnp.int32),
            jax.ShapeDtypeStruct((n, L, 4 * K), jnp.float32),
        ],
    )(q2, reference_points, Wx, Wy, bx, by, W_attn, ba)
    return idx, val


def _sc_build_a(idx4, val4, zeros_hbm, n, L, HW):
    rows_per_tile = L // (NC * NS) * 2
    del rows_per_tile
    words_per_tile = L * HW // NS
    n_rounds = n // NC
    chunks = idx4.shape[2]
    mesh = plsc.VectorSubcoreMesh(core_axis_name="c", subcore_axis_name="s")

    GRP = 16

    @functools.partial(
        pl.kernel,
        out_type=jax.ShapeDtypeStruct((n, L * HW), jnp.float32),
        mesh=mesh,
        scratch_types=[
            pltpu.VMEM((chunks, 128), jnp.int32),
            pltpu.VMEM((chunks, 128), jnp.float32),
            pltpu.VMEM_SHARED((L * HW,), jnp.float32),
            pltpu.SemaphoreType.DMA,
        ],
    )
    def build(idx_hbm, val_hbm, z_hbm, a_hbm, idx_v, val_v, a_sp, ssem):
        cid = lax.axis_index("c")
        sid = lax.axis_index("s")
        for r in range(n_rounds):
            b = r * NC + cid
            pltpu.sync_copy(z_hbm, a_sp.at[pl.ds(sid * words_per_tile,
                                                 words_per_tile)])
            pltpu.sync_copy(idx_hbm.at[b, sid], idx_v)
            pltpu.sync_copy(val_hbm.at[b, sid], val_v)

            def body(gidx, carry):
                descs = []
                for i in range(GRP):
                    j = gidx * GRP + i
                    descs.append(pltpu.async_copy(
                        val_v.at[j], a_sp.at[idx_v.at[j]], ssem, add=True))
                for d in descs:
                    d.wait()
                return carry

            lax.fori_loop(0, chunks // GRP, body, 0)
            pltpu.sync_copy(
                a_sp.at[pl.ds(sid * words_per_tile, words_per_tile)],
                a_hbm.at[b, pl.ds(sid * words_per_tile, words_per_tile)])

    return build(idx4, val4, zeros_hbm)


def _matmul_body(a_ref, qf_ref, wo_ref, bo_ref, o_ref, a_v, *, Lc):
    dn = (((1,), (1,)), ((), ()))
    i = pl.program_id(0)
    j = pl.program_id(1)
    pltpu.sync_copy(a_ref.at[i, pl.ds(j * Lc, Lc)], a_v)
    feat = qf_ref[0]
    s = jnp.dot(a_v[...], feat, preferred_element_type=jnp.float32)
    o = jax.lax.dot_general(s, wo_ref[...], dn,
                            preferred_element_type=jnp.float32)
    o_ref[0] = o + bo_ref[...]


def _final_matmul(a, q2, W_out, b_out, n, L, e):
    bo = b_out.reshape(1, e)
    Lc = 256
    HW = a.shape[2]
    return pl.pallas_call(
        functools.partial(_matmul_body, Lc=Lc),
        grid=(n, L // Lc),
        in_specs=[
            pl.BlockSpec(memory_space=pltpu.MemorySpace.HBM),
            pl.BlockSpec((1, L, e), lambda i, j: (i, 0, 0)),
            pl.BlockSpec((e, e), lambda i, j: (0, 0)),
            pl.BlockSpec((1, e), lambda i, j: (0, 0)),
        ],
        out_specs=pl.BlockSpec((1, Lc, e), lambda i, j: (i, j, 0)),
        out_shape=jax.ShapeDtypeStruct((n, L, e), jnp.float32),
        scratch_shapes=[pltpu.VMEM((Lc, HW), jnp.float32)],
    )(a, q2, W_out, bo)


def kernel(query, reference_points, W_off, b_off, W_attn, b_attn, W_out,
           b_out):
    n, h, w, e = query.shape
    L = h * w
    HW = h * w
    q2 = query.reshape(n, L, e)

    idx, val = _corner_lists(q2, reference_points, W_off, b_off, W_attn,
                             b_attn, h, w, L, e)
    idx4 = idx.reshape(n, NS, (L // NS) * 4 * K // 128, 128)
    val4 = val.reshape(n, NS, (L // NS) * 4 * K // 128, 128)
    zeros_hbm = jnp.zeros((L * HW // NS,), jnp.float32)

    a = _sc_build_a(idx4, val4, zeros_hbm, n, L, HW)
    out = _final_matmul(a.reshape(n, L, HW), q2, W_out, b_out, n, L, e)
    return out.reshape(n, h, w, e)


# speedup vs baseline: 1.2128x; 1.2128x over previous
import functools

import jax
import jax.numpy as jnp
from jax import lax
from jax.experimental import pallas as pl
from jax.experimental.pallas import tpu as pltpu
from jax.experimental.pallas import tpu_sc as plsc

NH = 8
NP = 4
K = NH * NP
NC = 2
NS = 16


def _corners_body(q_ref, rp_ref, wx_ref, wy_ref, bx_ref, by_ref, wa_ref,
                  ba_ref, oi_ref, ov_ref, *, W, H, L):
    dn = (((1,), (1,)), ((), ()))
    q = q_ref[0]
    rp = rp_ref[0]

    offx = jax.lax.dot_general(q, wx_ref[...], dn,
                               preferred_element_type=jnp.float32)
    offx = offx + bx_ref[...]
    offy = jax.lax.dot_general(q, wy_ref[...], dn,
                               preferred_element_type=jnp.float32)
    offy = offy + by_ref[...]
    logits = jax.lax.dot_general(q, wa_ref[...], dn,
                                 preferred_element_type=jnp.float32)
    logits = logits + ba_ref[...]

    m = jnp.max(logits, axis=1, keepdims=True)
    ex = jnp.exp(logits - m)
    gi = jax.lax.broadcasted_iota(jnp.int32, (K, K), 0) // NP
    gj = jax.lax.broadcasted_iota(jnp.int32, (K, K), 1) // NP
    G = (gi == gj).astype(jnp.float32)
    denom = jnp.dot(ex, G, preferred_element_type=jnp.float32)
    aw = ex / denom * (1.0 / NH)

    locx = rp[:, 0:1] + offx
    locy = rp[:, 1:2] + offy
    ix = locx * float(W) - 0.5
    iy = locy * float(H) - 0.5

    x0f = jnp.floor(ix)
    y0f = jnp.floor(iy)
    fx = ix - x0f
    fy = iy - y0f
    x1f = x0f + 1.0
    y1f = y0f + 1.0

    vx0 = ((x0f >= 0.0) & (x0f <= W - 1.0)).astype(jnp.float32)
    vx1 = ((x1f >= 0.0) & (x1f <= W - 1.0)).astype(jnp.float32)
    vy0 = ((y0f >= 0.0) & (y0f <= H - 1.0)).astype(jnp.float32)
    vy1 = ((y1f >= 0.0) & (y1f <= H - 1.0)).astype(jnp.float32)

    x0 = jnp.clip(x0f, 0.0, W - 1.0).astype(jnp.int32)
    x1 = jnp.clip(x1f, 0.0, W - 1.0).astype(jnp.int32)
    y0 = jnp.clip(y0f, 0.0, H - 1.0).astype(jnp.int32)
    y1 = jnp.clip(y1f, 0.0, H - 1.0).astype(jnp.int32)

    base = jax.lax.broadcasted_iota(jnp.int32, (L, 1), 0) * (H * W)
    g00 = base + y0 * W + x0
    g01 = base + y0 * W + x1
    g10 = base + y1 * W + x0
    g11 = base + y1 * W + x1

    w00 = aw * (1.0 - fy) * (1.0 - fx) * (vy0 * vx0)
    w01 = aw * (1.0 - fy) * fx * (vy0 * vx1)
    w10 = aw * fy * (1.0 - fx) * (vy1 * vx0)
    w11 = aw * fy * fx * (vy1 * vx1)

    oi_ref[0] = jnp.concatenate([g00, g01, g10, g11], axis=1)
    ov_ref[0] = jnp.concatenate([w00, w01, w10, w11], axis=1)


def _corner_lists(q2, reference_points, W_off, b_off, W_attn, b_attn,
                  h, w, L, e):
    Wx = W_off[0::2]
    Wy = W_off[1::2]
    bx = b_off[0::2].reshape(1, K)
    by = b_off[1::2].reshape(1, K)
    ba = b_attn.reshape(1, K)
    n = q2.shape[0]
    idx, val = pl.pallas_call(
        functools.partial(_corners_body, W=w, H=h, L=L),
        grid=(n,),
        in_specs=[
            pl.BlockSpec((1, L, e), lambda i: (i, 0, 0)),
            pl.BlockSpec((1, L, 2), lambda i: (i, 0, 0)),
            pl.BlockSpec((K, e), lambda i: (0, 0)),
            pl.BlockSpec((K, e), lambda i: (0, 0)),
            pl.BlockSpec((1, K), lambda i: (0, 0)),
            pl.BlockSpec((1, K), lambda i: (0, 0)),
            pl.BlockSpec((K, e), lambda i: (0, 0)),
            pl.BlockSpec((1, K), lambda i: (0, 0)),
        ],
        out_specs=[
            pl.BlockSpec((1, L, 4 * K), lambda i: (i, 0, 0)),
            pl.BlockSpec((1, L, 4 * K), lambda i: (i, 0, 0)),
        ],
        out_shape=[
            jax.ShapeDtypeStruct((n, L, 4 * K), jnp.int32),
            jax.ShapeDtypeStruct((n, L, 4 * K), jnp.float32),
        ],
    )(q2, reference_points, Wx, Wy, bx, by, W_attn, ba)
    return idx, val


def _sc_build_a(idx4, val4, zeros_hbm, n, L, HW):
    rows_per_tile = L // (NC * NS) * 2
    del rows_per_tile
    words_per_tile = L * HW // NS
    n_rounds = n // NC
    chunks = idx4.shape[2]
    mesh = plsc.VectorSubcoreMesh(core_axis_name="c", subcore_axis_name="s")

    GRP = 16

    @functools.partial(
        pl.kernel,
        out_type=jax.ShapeDtypeStruct((n, L * HW), jnp.float32),
        mesh=mesh,
        scratch_types=[
            pltpu.VMEM((chunks, 128), jnp.int32),
            pltpu.VMEM((chunks, 128), jnp.float32),
            pltpu.VMEM_SHARED((L * HW,), jnp.float32),
            pltpu.SemaphoreType.DMA,
        ],
    )
    def build(idx_hbm, val_hbm, z_hbm, a_hbm, idx_v, val_v, a_sp, ssem):
        cid = lax.axis_index("c")
        sid = lax.axis_index("s")
        for r in range(n_rounds):
            b = r * NC + cid
            pltpu.sync_copy(z_hbm, a_sp.at[pl.ds(sid * words_per_tile,
                                                 words_per_tile)])
            pltpu.sync_copy(idx_hbm.at[b, sid], idx_v)
            pltpu.sync_copy(val_hbm.at[b, sid], val_v)

            def body(gidx, carry):
                descs = []
                for i in range(GRP):
                    j = gidx * GRP + i
                    descs.append(pltpu.async_copy(
                        val_v.at[j], a_sp.at[idx_v.at[j]], ssem, add=True))
                for d in descs:
                    d.wait()
                return carry

            lax.fori_loop(0, chunks // GRP, body, 0)
            pltpu.sync_copy(
                a_sp.at[pl.ds(sid * words_per_tile, words_per_tile)],
                a_hbm.at[b, pl.ds(sid * words_per_tile, words_per_tile)])

    return build(idx4, val4, zeros_hbm)


def _matmul_body(a_ref, qf_ref, wo_ref, bo_ref, o_ref):
    dn = (((1,), (1,)), ((), ()))
    a = a_ref[0]
    feat = qf_ref[0]
    s = jnp.dot(a, feat, preferred_element_type=jnp.float32)
    o = jax.lax.dot_general(s, wo_ref[...], dn,
                            preferred_element_type=jnp.float32)
    o_ref[0] = o + bo_ref[...]


def _final_matmul(a, q2, W_out, b_out, n, L, e):
    bo = b_out.reshape(1, e)
    Lc = 512
    return pl.pallas_call(
        _matmul_body,
        grid=(n, L // Lc),
        in_specs=[
            pl.BlockSpec((1, Lc, L), lambda i, j: (i, j, 0)),
            pl.BlockSpec((1, L, e), lambda i, j: (i, 0, 0)),
            pl.BlockSpec((e, e), lambda i, j: (0, 0)),
            pl.BlockSpec((1, e), lambda i, j: (0, 0)),
        ],
        out_specs=pl.BlockSpec((1, Lc, e), lambda i, j: (i, j, 0)),
        out_shape=jax.ShapeDtypeStruct((n, L, e), jnp.float32),
    )(a, q2, W_out, bo)


def kernel(query, reference_points, W_off, b_off, W_attn, b_attn, W_out,
           b_out):
    n, h, w, e = query.shape
    L = h * w
    HW = h * w
    q2 = query.reshape(n, L, e)

    idx, val = _corner_lists(q2, reference_points, W_off, b_off, W_attn,
                             b_attn, h, w, L, e)
    idx4 = idx.reshape(n, NS, (L // NS) * 4 * K // 128, 128)
    val4 = val.reshape(n, NS, (L // NS) * 4 * K // 128, 128)
    zeros_hbm = jnp.zeros((L * HW // NS,), jnp.float32)

    a = _sc_build_a(idx4, val4, zeros_hbm, n, L, HW)
    out = _final_matmul(a.reshape(n, L, HW), q2, W_out, b_out, n, L, e)
    return out.reshape(n, h, w, e)


# speedup vs baseline: 1.2451x; 1.0266x over previous
import functools

import jax
import jax.numpy as jnp
from jax import lax
from jax.experimental import pallas as pl
from jax.experimental.pallas import tpu as pltpu
from jax.experimental.pallas import tpu_sc as plsc

NH = 8
NP = 4
K = NH * NP
NC = 2
NS = 16


def _corners_body(q_ref, rp_ref, wx_ref, wy_ref, bx_ref, by_ref, wa_ref,
                  ba_ref, oi_ref, ov_ref, *, W, H, L):
    dn = (((1,), (1,)), ((), ()))
    q = q_ref[0]
    rp = rp_ref[0]

    offx = jax.lax.dot_general(q, wx_ref[...], dn,
                               preferred_element_type=jnp.float32)
    offx = offx + bx_ref[...]
    offy = jax.lax.dot_general(q, wy_ref[...], dn,
                               preferred_element_type=jnp.float32)
    offy = offy + by_ref[...]
    logits = jax.lax.dot_general(q, wa_ref[...], dn,
                                 preferred_element_type=jnp.float32)
    logits = logits + ba_ref[...]

    m = jnp.max(logits, axis=1, keepdims=True)
    ex = jnp.exp(logits - m)
    gi = jax.lax.broadcasted_iota(jnp.int32, (K, K), 0) // NP
    gj = jax.lax.broadcasted_iota(jnp.int32, (K, K), 1) // NP
    G = (gi == gj).astype(jnp.float32)
    denom = jnp.dot(ex, G, preferred_element_type=jnp.float32)
    aw = ex / denom * (1.0 / NH)

    locx = rp[:, 0:1] + offx
    locy = rp[:, 1:2] + offy
    ix = locx * float(W) - 0.5
    iy = locy * float(H) - 0.5

    x0f = jnp.floor(ix)
    y0f = jnp.floor(iy)
    fx = ix - x0f
    fy = iy - y0f
    x1f = x0f + 1.0
    y1f = y0f + 1.0

    vx0 = ((x0f >= 0.0) & (x0f <= W - 1.0)).astype(jnp.float32)
    vx1 = ((x1f >= 0.0) & (x1f <= W - 1.0)).astype(jnp.float32)
    vy0 = ((y0f >= 0.0) & (y0f <= H - 1.0)).astype(jnp.float32)
    vy1 = ((y1f >= 0.0) & (y1f <= H - 1.0)).astype(jnp.float32)

    x0 = jnp.clip(x0f, 0.0, W - 1.0).astype(jnp.int32)
    x1 = jnp.clip(x1f, 0.0, W - 1.0).astype(jnp.int32)
    y0 = jnp.clip(y0f, 0.0, H - 1.0).astype(jnp.int32)
    y1 = jnp.clip(y1f, 0.0, H - 1.0).astype(jnp.int32)

    base = jax.lax.broadcasted_iota(jnp.int32, (L, 1), 0) * (H * W)
    g00 = base + y0 * W + x0
    g01 = base + y0 * W + x1
    g10 = base + y1 * W + x0
    g11 = base + y1 * W + x1

    w00 = aw * (1.0 - fy) * (1.0 - fx) * (vy0 * vx0)
    w01 = aw * (1.0 - fy) * fx * (vy0 * vx1)
    w10 = aw * fy * (1.0 - fx) * (vy1 * vx0)
    w11 = aw * fy * fx * (vy1 * vx1)

    oi_ref[0] = jnp.concatenate([g00, g01, g10, g11], axis=1)
    ov_ref[0] = jnp.concatenate([w00, w01, w10, w11], axis=1)


def _corner_lists(q2, reference_points, W_off, b_off, W_attn, b_attn,
                  h, w, L, e):
    Wx = W_off[0::2]
    Wy = W_off[1::2]
    bx = b_off[0::2].reshape(1, K)
    by = b_off[1::2].reshape(1, K)
    ba = b_attn.reshape(1, K)
    n = q2.shape[0]
    idx, val = pl.pallas_call(
        functools.partial(_corners_body, W=w, H=h, L=L),
        grid=(n,),
        in_specs=[
            pl.BlockSpec((1, L, e), lambda i: (i, 0, 0)),
            pl.BlockSpec((1, L, 2), lambda i: (i, 0, 0)),
            pl.BlockSpec((K, e), lambda i: (0, 0)),
            pl.BlockSpec((K, e), lambda i: (0, 0)),
            pl.BlockSpec((1, K), lambda i: (0, 0)),
            pl.BlockSpec((1, K), lambda i: (0, 0)),
            pl.BlockSpec((K, e), lambda i: (0, 0)),
            pl.BlockSpec((1, K), lambda i: (0, 0)),
        ],
        out_specs=[
            pl.BlockSpec((1, L, 4 * K), lambda i: (i, 0, 0)),
            pl.BlockSpec((1, L, 4 * K), lambda i: (i, 0, 0)),
        ],
        out_shape=[
            jax.ShapeDtypeStruct((n, L, 4 * K), jnp.int32),
            jax.ShapeDtypeStruct((n, L, 4 * K), jnp.float32),
        ],
    )(q2, reference_points, Wx, Wy, bx, by, W_attn, ba)
    return idx, val


def _sc_build_a(idx4, val4, zeros_hbm, n, L, HW):
    rows_per_tile = L // (NC * NS) * 2
    del rows_per_tile
    words_per_tile = L * HW // NS
    n_rounds = n // NC
    chunks = idx4.shape[2]
    mesh = plsc.VectorSubcoreMesh(core_axis_name="c", subcore_axis_name="s")

    GRP = 16

    @functools.partial(
        pl.kernel,
        out_type=jax.ShapeDtypeStruct((n, L * HW), jnp.float32),
        mesh=mesh,
        scratch_types=[
            pltpu.VMEM((chunks, 128), jnp.int32),
            pltpu.VMEM((chunks, 128), jnp.float32),
            pltpu.VMEM_SHARED((L * HW,), jnp.float32),
            pltpu.SemaphoreType.DMA,
        ],
    )
    def build(idx_hbm, val_hbm, z_hbm, a_hbm, idx_v, val_v, a_sp, ssem):
        cid = lax.axis_index("c")
        sid = lax.axis_index("s")
        for r in range(n_rounds):
            b = r * NC + cid
            pltpu.sync_copy(z_hbm, a_sp.at[pl.ds(sid * words_per_tile,
                                                 words_per_tile)])
            pltpu.sync_copy(idx_hbm.at[b, sid], idx_v)
            pltpu.sync_copy(val_hbm.at[b, sid], val_v)

            def body(gidx, carry):
                descs = []
                for i in range(GRP):
                    j = gidx * GRP + i
                    descs.append(pltpu.async_copy(
                        val_v.at[j], a_sp.at[idx_v.at[j]], ssem, add=True))
                for d in descs:
                    d.wait()
                return carry

            lax.fori_loop(0, chunks // GRP, body, 0)
            pltpu.sync_copy(
                a_sp.at[pl.ds(sid * words_per_tile, words_per_tile)],
                a_hbm.at[b, pl.ds(sid * words_per_tile, words_per_tile)])

    return build(idx4, val4, zeros_hbm)


def _matmul_body(a_ref, qf_ref, wo_ref, bo_ref, o_ref):
    dn = (((1,), (1,)), ((), ()))
    a = a_ref[0]
    feat = qf_ref[0]
    s = jnp.dot(a, feat, preferred_element_type=jnp.float32)
    o = jax.lax.dot_general(s, wo_ref[...], dn,
                            preferred_element_type=jnp.float32)
    o_ref[0] = o + bo_ref[...]


def _final_matmul(a, q2, W_out, b_out, n, L, e):
    bo = b_out.reshape(1, e)
    Lc = 1024
    return pl.pallas_call(
        _matmul_body,
        grid=(n, L // Lc),
        in_specs=[
            pl.BlockSpec((1, Lc, L), lambda i, j: (i, j, 0)),
            pl.BlockSpec((1, L, e), lambda i, j: (i, 0, 0)),
            pl.BlockSpec((e, e), lambda i, j: (0, 0)),
            pl.BlockSpec((1, e), lambda i, j: (0, 0)),
        ],
        out_specs=pl.BlockSpec((1, Lc, e), lambda i, j: (i, j, 0)),
        out_shape=jax.ShapeDtypeStruct((n, L, e), jnp.float32),
    )(a, q2, W_out, bo)


def kernel(query, reference_points, W_off, b_off, W_attn, b_attn, W_out,
           b_out):
    n, h, w, e = query.shape
    L = h * w
    HW = h * w
    q2 = query.reshape(n, L, e)

    idx, val = _corner_lists(q2, reference_points, W_off, b_off, W_attn,
                             b_attn, h, w, L, e)
    idx4 = idx.reshape(n, NS, (L // NS) * 4 * K // 128, 128)
    val4 = val.reshape(n, NS, (L // NS) * 4 * K // 128, 128)
    zeros_hbm = jnp.zeros((L * HW // NS,), jnp.float32)

    a = _sc_build_a(idx4, val4, zeros_hbm, n, L, HW)
    out = _final_matmul(a.reshape(n, L, HW), q2, W_out, b_out, n, L, e)
    return out.reshape(n, h, w, e)
